# baseline (speedup 1.0000x reference)
import functools

import jax
import jax.numpy as jnp
from jax.experimental import pallas as pl
from jax.experimental.pallas import tpu as pltpu
from jax.experimental.pallas import tpu_sc as plsc

ALPHA = 0.1
K = 10
NS = 16
EB = 128
NCH = 160
RSUB = 128
NPAD = 10240
CH = 32

_SC_PARAMS = pltpu.CompilerParams(use_tc_tiling_on_sc=False)
_MESH = dict(core_axis_name="c", subcore_axis_name="s")


def _sc_hist(dst2d):
    hpt = NCH // 2
    rpt = NPAD // NS

    @functools.partial(
        pl.kernel,
        out_type=jax.ShapeDtypeStruct((2, NPAD, 16), jnp.float32),
        mesh=plsc.VectorSubcoreMesh(**_MESH),
        compiler_params=_SC_PARAMS,
        scratch_types=[
            pltpu.VMEM_SHARED((NPAD, 16), jnp.float32),
            pltpu.VMEM((rpt, 16), jnp.float32),
            pltpu.VMEM((EB, 16), jnp.float32),
            pltpu.VMEM((EB,), jnp.int32),
        ],
    )
    def hist_kernel(dst_hbm, out_hbm, acc, rowbuf, ones_b, didx):
        core = jax.lax.axis_index("c")
        tile = jax.lax.axis_index("s")
        zeros16 = jnp.zeros((16,), jnp.float32)
        ones16 = jnp.ones((16,), jnp.float32)

        @pl.loop(0, rpt)
        def _(j):
            rowbuf[j, :] = zeros16

        pltpu.sync_copy(rowbuf, acc.at[pl.ds(tile * rpt, rpt)])

        @pl.loop(0, EB)
        def _(j):
            ones_b[j, :] = ones16

        plsc.subcore_barrier()
        c0 = tile * NCH + core * hpt

        @pl.loop(0, hpt)
        def _(i):
            pltpu.sync_copy(dst_hbm.at[c0 + i], didx)
            pltpu.sync_copy(ones_b, acc.at[didx], add=True)

        plsc.subcore_barrier()
        pltpu.sync_copy(acc.at[pl.ds(tile * rpt, rpt)], rowbuf)
        pltpu.sync_copy(rowbuf, out_hbm.at[core].at[pl.ds(tile * rpt, rpt)])

    return hist_kernel(dst2d)


def _tc_prep(x, W1, b1, W2, b2, deg16):
    n = x.shape[0]
    c = W2.shape[1]

    blk = 1280
    grid = n // blk

    def body(x_ref, w1_ref, b1_ref, w2_ref, b2_ref, deg_ref,
             ztlo_ref, zthi_ref, u_ref, htlo_ref, hthi_ref, din_ref, ah_ref):
        h1 = jnp.maximum(
            jnp.dot(x_ref[...], w1_ref[...], preferred_element_type=jnp.float32)
            + b1_ref[...], 0.0)
        h = jnp.dot(h1, w2_ref[...], preferred_element_type=jnp.float32) + b2_ref[...]
        deg = deg_ref[...][0, :, 0:1] + deg_ref[...][1, :, 0:1] + 1.0
        dinv = jax.lax.rsqrt(deg)
        zt = dinv * h
        ztlo_ref[...] = zt[:, :CH]
        zthi_ref[...] = zt[:, CH:]
        htlo_ref[...] = ALPHA * zt[:, :CH]
        hthi_ref[...] = ALPHA * zt[:, CH:]
        u_ref[...] = jnp.broadcast_to((1.0 - ALPHA) * dinv * dinv, (blk, CH))
        din_ref[...] = jnp.broadcast_to((1.0 - ALPHA) * dinv, h.shape)
        ah_ref[...] = ALPHA * h

    f = jnp.float32
    row = lambda i: (i, 0)
    bs_h = pl.BlockSpec((blk, CH), row)
    bs_c = pl.BlockSpec((blk, c), row)
    return pl.pallas_call(
        body,
        grid=(grid,),
        in_specs=[
            pl.BlockSpec((blk, x.shape[1]), row),
            pl.BlockSpec(W1.shape, lambda i: (0, 0)),
            pl.BlockSpec((1, b1.shape[0]), lambda i: (0, 0)),
            pl.BlockSpec(W2.shape, lambda i: (0, 0)),
            pl.BlockSpec((1, b2.shape[0]), lambda i: (0, 0)),
            pl.BlockSpec((2, blk, 16), lambda i: (0, i, 0)),
        ],
        out_specs=[bs_h, bs_h, bs_h, bs_h, bs_h, bs_c, bs_c],
        out_shape=(
            jax.ShapeDtypeStruct((n, CH), f),
            jax.ShapeDtypeStruct((n, CH), f),
            jax.ShapeDtypeStruct((n, CH), f),
            jax.ShapeDtypeStruct((n, CH), f),
            jax.ShapeDtypeStruct((n, CH), f),
            jax.ShapeDtypeStruct((n, c), f),
            jax.ShapeDtypeStruct((n, c), f),
        ),
    )(x, W1, b1.reshape(1, -1), W2, b2.reshape(1, -1), deg16)


def _sc_prop(zt0f, src2d, dst2d, u32, htf):
    rpt = NPAD // NS
    nsub = rpt // RSUB

    @functools.partial(
        pl.kernel,
        out_type=(jax.ShapeDtypeStruct((2 * NPAD, CH), jnp.float32),
                  jax.ShapeDtypeStruct((2 * NPAD, CH), jnp.float32)),
        mesh=plsc.VectorSubcoreMesh(**_MESH),
        compiler_params=_SC_PARAMS,
        scratch_types=[
            pltpu.VMEM_SHARED((NPAD, CH), jnp.float32),
            pltpu.VMEM((NCH, EB), jnp.int32),
            pltpu.VMEM((NCH, EB), jnp.int32),
            pltpu.VMEM((RSUB, CH), jnp.float32),
            pltpu.VMEM((RSUB, CH), jnp.float32),
            pltpu.VMEM((EB, CH), jnp.float32),
            pltpu.VMEM((EB, CH), jnp.float32),
            pltpu.VMEM((RSUB, CH), jnp.float32),
            pltpu.SemaphoreType.DMA,
            pltpu.SemaphoreType.DMA,
            pltpu.SemaphoreType.DMA,
            pltpu.SemaphoreType.DMA,
        ],
    )
    def prop_kernel(zt0_hbm, src_hbm, dst_hbm, u_hbm, ht_hbm, sout_hbm,
                    ztb_hbm, agg, sidx, didx, ubuf, hbuf, rows_a, rows_b, abuf,
                    gsa, gsb, ssa, ssb):
        core = jax.lax.axis_index("c")
        tile = jax.lax.axis_index("s")
        r0 = tile * rpt
        h0 = core * NPAD + r0

        pltpu.sync_copy(src_hbm.at[pl.ds(tile * NCH, NCH)], sidx)
        pltpu.sync_copy(dst_hbm.at[pl.ds(tile * NCH, NCH)], didx)
        cbase = jnp.broadcast_to((core * NPAD).astype(jnp.int32), (16,))

        @pl.loop(0, NCH)
        def _(i):
            for cc in range(EB // 16):
                csl = pl.ds(cc * 16, 16)
                sidx[i, csl] = sidx[i, csl] + cbase
        for sub in range(nsub):
            ssp = pl.ds(r0 + sub * RSUB, RSUB)
            pltpu.sync_copy(zt0_hbm.at[pl.ds(h0 + sub * RSUB, RSUB)], abuf)
            pltpu.sync_copy(abuf, ztb_hbm.at[pl.ds(h0 + sub * RSUB, RSUB)])
            pltpu.sync_copy(abuf, agg.at[ssp])
        plsc.subcore_barrier()

        def g_start(i, buf, sem):
            pltpu.async_copy(ztb_hbm.at[sidx.at[i]], buf, sem)

        def g_wait(buf, sem):
            pltpu.make_async_copy(ztb_hbm.at[sidx.at[0]], buf, sem).wait()

        def s_start(i, buf, sem):
            pltpu.async_copy(buf, agg.at[didx.at[i]], sem, add=True)

        def s_wait(buf, sem):
            pltpu.make_async_copy(buf, agg.at[didx.at[0]], sem).wait()

        npairs = NCH // 2

        def edge_pass():
            g_start(0, rows_a, gsa)

            @pl.loop(0, npairs)
            def _(p):
                i0 = 2 * p
                g_wait(rows_a, gsa)

                @pl.when(p > 0)
                def _():
                    s_wait(rows_b, ssb)

                g_start(i0 + 1, rows_b, gsb)
                s_start(i0, rows_a, ssa)
                g_wait(rows_b, gsb)
                s_wait(rows_a, ssa)

                @pl.when(p < npairs - 1)
                def _():
                    g_start(i0 + 2, rows_a, gsa)

                s_start(i0 + 1, rows_b, ssb)

            s_wait(rows_b, ssb)

        @pl.loop(0, K - 1)
        def _(k):
            edge_pass()
            plsc.subcore_barrier()
            for sub in range(nsub):
                ssp = pl.ds(r0 + sub * RSUB, RSUB)
                pltpu.sync_copy(agg.at[ssp], abuf)
                pltpu.sync_copy(u_hbm.at[pl.ds(r0 + sub * RSUB, RSUB)], ubuf)
                pltpu.sync_copy(ht_hbm.at[pl.ds(h0 + sub * RSUB, RSUB)], hbuf)

                @pl.loop(0, RSUB)
                def _(j):
                    for cc in range(CH // 16):
                        csl = pl.ds(cc * 16, 16)
                        abuf[j, csl] = (ubuf[j, csl] * abuf[j, csl]
                                        + hbuf[j, csl])

                pltpu.sync_copy(abuf, ztb_hbm.at[pl.ds(h0 + sub * RSUB, RSUB)])
                pltpu.sync_copy(abuf, agg.at[ssp])
            plsc.subcore_barrier()

        edge_pass()
        plsc.subcore_barrier()
        for sub in range(nsub):
            pltpu.sync_copy(agg.at[pl.ds(r0 + sub * RSUB, RSUB)], abuf)
            pltpu.sync_copy(abuf, sout_hbm.at[pl.ds(h0 + sub * RSUB, RSUB)])

    return prop_kernel(zt0f, src2d, dst2d, u32, htf)[0]


def _tc_finish(sf, din64, ah64):
    n, c = din64.shape

    def body(s_ref, din_ref, ah_ref, o_ref):
        s = jnp.concatenate([s_ref[...][:n], s_ref[...][n:]], axis=1)
        z = din_ref[...] * s + ah_ref[...]
        m = jnp.max(z, axis=1, keepdims=True)
        lse = jnp.log(jnp.sum(jnp.exp(z - m), axis=1, keepdims=True)) + m
        o_ref[...] = z - lse

    return pl.pallas_call(
        body, out_shape=jax.ShapeDtypeStruct((n, c), jnp.float32),
    )(sf, din64, ah64)


def kernel(x, edge_index, W1, b1, W2, b2):
    n = x.shape[0]
    e = edge_index.shape[1]
    epad = NS * NCH * EB
    xp = jnp.pad(x, ((0, NPAD - n), (0, 0)))
    pad = jnp.full((epad - e,), NPAD - 1, jnp.int32)
    src2d = jnp.concatenate([edge_index[0], pad]).reshape(NS * NCH, EB)
    dst2d = jnp.concatenate([edge_index[1], pad]).reshape(NS * NCH, EB)
    deg16 = _sc_hist(dst2d)
    ztlo, zthi, u32, htlo, hthi, din64, ah64 = _tc_prep(xp, W1, b1, W2, b2, deg16)
    zt0f = jnp.concatenate([ztlo, zthi], axis=0)
    htf = jnp.concatenate([htlo, hthi], axis=0)
    sf = _sc_prop(zt0f, src2d, dst2d, u32, htf)
    return _tc_finish(sf, din64, ah64)[:n]


# speedup vs baseline: 2.2243x; 2.2243x over previous
import functools

import jax
import jax.numpy as jnp
from jax.experimental import pallas as pl
from jax.experimental.pallas import tpu as pltpu
from jax.experimental.pallas import tpu_sc as plsc

ALPHA = 0.1
K = 10
NS = 16
EB = 128
NCH = 160
RSUB = 128
NPAD = 10240
CH = 32

_SC_PARAMS = pltpu.CompilerParams(use_tc_tiling_on_sc=False)
_MESH = dict(core_axis_name="c", subcore_axis_name="s")


def _sc_hist(dst2d):
    hpt = NCH // 2
    rpt = NPAD // NS

    @functools.partial(
        pl.kernel,
        out_type=jax.ShapeDtypeStruct((2, NPAD, 16), jnp.float32),
        mesh=plsc.VectorSubcoreMesh(**_MESH),
        compiler_params=_SC_PARAMS,
        scratch_types=[
            pltpu.VMEM_SHARED((NPAD, 16), jnp.float32),
            pltpu.VMEM((rpt, 16), jnp.float32),
            pltpu.VMEM((EB, 16), jnp.float32),
            pltpu.VMEM((EB,), jnp.int32),
        ],
    )
    def hist_kernel(dst_hbm, out_hbm, acc, rowbuf, ones_b, didx):
        core = jax.lax.axis_index("c")
        tile = jax.lax.axis_index("s")
        zeros16 = jnp.zeros((16,), jnp.float32)
        ones16 = jnp.ones((16,), jnp.float32)

        @pl.loop(0, rpt)
        def _(j):
            rowbuf[j, :] = zeros16

        pltpu.sync_copy(rowbuf, acc.at[pl.ds(tile * rpt, rpt)])

        @pl.loop(0, EB)
        def _(j):
            ones_b[j, :] = ones16

        plsc.subcore_barrier()
        c0 = tile * NCH + core * hpt

        @pl.loop(0, hpt)
        def _(i):
            pltpu.sync_copy(dst_hbm.at[c0 + i], didx)
            pltpu.sync_copy(ones_b, acc.at[didx], add=True)

        plsc.subcore_barrier()
        pltpu.sync_copy(acc.at[pl.ds(tile * rpt, rpt)], rowbuf)
        pltpu.sync_copy(rowbuf, out_hbm.at[core].at[pl.ds(tile * rpt, rpt)])

    return hist_kernel(dst2d)


def _tc_mlp(x, W1, b1, W2, b2):
    n = x.shape[0]
    c = W2.shape[1]
    blk = 1280

    def body(x_ref, w1_ref, b1_ref, w2_ref, b2_ref, h_ref, ah_ref):
        h1 = jnp.maximum(
            jnp.dot(x_ref[...], w1_ref[...], preferred_element_type=jnp.float32)
            + b1_ref[...], 0.0)
        h = jnp.dot(h1, w2_ref[...], preferred_element_type=jnp.float32) + b2_ref[...]
        h_ref[...] = h
        ah_ref[...] = ALPHA * h

    f = jnp.float32
    row = lambda i: (i, 0)
    bs_c = pl.BlockSpec((blk, c), row)
    return pl.pallas_call(
        body,
        grid=(n // blk,),
        in_specs=[
            pl.BlockSpec((blk, x.shape[1]), row),
            pl.BlockSpec(W1.shape, lambda i: (0, 0)),
            pl.BlockSpec((1, b1.shape[0]), lambda i: (0, 0)),
            pl.BlockSpec(W2.shape, lambda i: (0, 0)),
            pl.BlockSpec((1, b2.shape[0]), lambda i: (0, 0)),
        ],
        out_specs=[bs_c, bs_c],
        out_shape=(jax.ShapeDtypeStruct((n, c), f),
                   jax.ShapeDtypeStruct((n, c), f)),
    )(x, W1, b1.reshape(1, -1), W2, b2.reshape(1, -1))


def _tc_scale(h, deg16):
    n, c = h.shape
    blk = 1280

    def body(h_ref, deg_ref, ztlo_ref, zthi_ref, u_ref, htlo_ref, hthi_ref,
             din_ref):
        h = h_ref[...]
        deg = deg_ref[...][0, :, 0:1] + deg_ref[...][1, :, 0:1] + 1.0
        dinv = jax.lax.rsqrt(deg)
        zt = dinv * h
        ztlo_ref[...] = zt[:, :CH]
        zthi_ref[...] = zt[:, CH:]
        htlo_ref[...] = ALPHA * zt[:, :CH]
        hthi_ref[...] = ALPHA * zt[:, CH:]
        u_ref[...] = jnp.broadcast_to((1.0 - ALPHA) * dinv * dinv, (blk, CH))
        din_ref[...] = jnp.broadcast_to((1.0 - ALPHA) * dinv, (blk, c))

    f = jnp.float32
    row = lambda i: (i, 0)
    bs_h = pl.BlockSpec((blk, CH), row)
    bs_c = pl.BlockSpec((blk, c), row)
    return pl.pallas_call(
        body,
        grid=(n // blk,),
        in_specs=[bs_c, pl.BlockSpec((2, blk, 16), lambda i: (0, i, 0))],
        out_specs=[bs_h, bs_h, bs_h, bs_h, bs_h, bs_c],
        out_shape=(
            jax.ShapeDtypeStruct((n, CH), f),
            jax.ShapeDtypeStruct((n, CH), f),
            jax.ShapeDtypeStruct((n, CH), f),
            jax.ShapeDtypeStruct((n, CH), f),
            jax.ShapeDtypeStruct((n, CH), f),
            jax.ShapeDtypeStruct((n, c), f),
        ),
    )(h, deg16)


def _sc_prop(zt0f, src2d, dst2d, u32, htf):
    rpt = NPAD // NS
    nsub = rpt // RSUB

    @functools.partial(
        pl.kernel,
        out_type=jax.ShapeDtypeStruct((2 * NPAD, CH), jnp.float32),
        mesh=plsc.VectorSubcoreMesh(**_MESH),
        compiler_params=_SC_PARAMS,
        scratch_types=[
            pltpu.VMEM_SHARED((NPAD, CH), jnp.float32),
            pltpu.VMEM_SHARED((NPAD, CH), jnp.float32),
            pltpu.VMEM((NCH, EB), jnp.int32),
            pltpu.VMEM((NCH, EB), jnp.int32),
            pltpu.VMEM((RSUB, CH), jnp.float32),
            pltpu.VMEM((RSUB, CH), jnp.float32),
            pltpu.VMEM((EB, CH), jnp.float32),
            pltpu.VMEM((EB, CH), jnp.float32),
            pltpu.VMEM((RSUB, CH), jnp.float32),
            pltpu.SemaphoreType.DMA,
            pltpu.SemaphoreType.DMA,
            pltpu.SemaphoreType.DMA,
            pltpu.SemaphoreType.DMA,
        ],
    )
    def prop_kernel(zt0_hbm, src_hbm, dst_hbm, u_hbm, ht_hbm, sout_hbm,
                    zt, agg, sidx, didx, ubuf, hbuf, rows_a, rows_b, abuf,
                    gsa, gsb, ssa, ssb):
        core = jax.lax.axis_index("c")
        tile = jax.lax.axis_index("s")
        r0 = tile * rpt
        h0 = core * NPAD + r0

        pltpu.sync_copy(src_hbm.at[pl.ds(tile * NCH, NCH)], sidx)
        pltpu.sync_copy(dst_hbm.at[pl.ds(tile * NCH, NCH)], didx)
        for sub in range(nsub):
            ssp = pl.ds(r0 + sub * RSUB, RSUB)
            pltpu.sync_copy(zt0_hbm.at[pl.ds(h0 + sub * RSUB, RSUB)], abuf)
            pltpu.sync_copy(abuf, zt.at[ssp])
            pltpu.sync_copy(abuf, agg.at[ssp])
        plsc.subcore_barrier()

        def g_start(i, buf, sem):
            pltpu.async_copy(zt.at[sidx.at[i]], buf, sem)

        def g_wait(buf, sem):
            pltpu.make_async_copy(zt.at[sidx.at[0]], buf, sem).wait()

        def s_start(i, buf, sem):
            pltpu.async_copy(buf, agg.at[didx.at[i]], sem, add=True)

        def s_wait(buf, sem):
            pltpu.make_async_copy(buf, agg.at[didx.at[0]], sem).wait()

        npairs = NCH // 2

        def edge_pass():
            g_start(0, rows_a, gsa)

            @pl.loop(0, npairs)
            def _(p):
                i0 = 2 * p
                g_wait(rows_a, gsa)

                @pl.when(p > 0)
                def _():
                    s_wait(rows_b, ssb)

                g_start(i0 + 1, rows_b, gsb)
                s_start(i0, rows_a, ssa)
                g_wait(rows_b, gsb)
                s_wait(rows_a, ssa)

                @pl.when(p < npairs - 1)
                def _():
                    g_start(i0 + 2, rows_a, gsa)

                s_start(i0 + 1, rows_b, ssb)

            s_wait(rows_b, ssb)

        @pl.loop(0, K - 1)
        def _(k):
            edge_pass()
            plsc.subcore_barrier()
            for sub in range(nsub):
                ssp = pl.ds(r0 + sub * RSUB, RSUB)
                pltpu.sync_copy(agg.at[ssp], abuf)
                pltpu.sync_copy(u_hbm.at[pl.ds(r0 + sub * RSUB, RSUB)], ubuf)
                pltpu.sync_copy(ht_hbm.at[pl.ds(h0 + sub * RSUB, RSUB)], hbuf)

                @pl.loop(0, RSUB)
                def _(j):
                    for cc in range(CH // 16):
                        csl = pl.ds(cc * 16, 16)
                        abuf[j, csl] = (ubuf[j, csl] * abuf[j, csl]
                                        + hbuf[j, csl])

                pltpu.sync_copy(abuf, zt.at[ssp])
                pltpu.sync_copy(abuf, agg.at[ssp])
            plsc.subcore_barrier()

        edge_pass()
        plsc.subcore_barrier()
        for sub in range(nsub):
            pltpu.sync_copy(agg.at[pl.ds(r0 + sub * RSUB, RSUB)], abuf)
            pltpu.sync_copy(abuf, sout_hbm.at[pl.ds(h0 + sub * RSUB, RSUB)])

    return prop_kernel(zt0f, src2d, dst2d, u32, htf)


def _tc_finish(sf, din64, ah64):
    n, c = din64.shape

    def body(s_ref, din_ref, ah_ref, o_ref):
        s = jnp.concatenate([s_ref[...][:n], s_ref[...][n:]], axis=1)
        z = din_ref[...] * s + ah_ref[...]
        m = jnp.max(z, axis=1, keepdims=True)
        lse = jnp.log(jnp.sum(jnp.exp(z - m), axis=1, keepdims=True)) + m
        o_ref[...] = z - lse

    return pl.pallas_call(
        body, out_shape=jax.ShapeDtypeStruct((n, c), jnp.float32),
    )(sf, din64, ah64)


def kernel(x, edge_index, W1, b1, W2, b2):
    n = x.shape[0]
    e = edge_index.shape[1]
    epad = NS * NCH * EB
    xp = jnp.pad(x, ((0, NPAD - n), (0, 0)))
    pad = jnp.full((epad - e,), NPAD - 1, jnp.int32)
    src2d = jnp.concatenate([edge_index[0], pad]).reshape(NS * NCH, EB)
    dst2d = jnp.concatenate([edge_index[1], pad]).reshape(NS * NCH, EB)
    deg16 = _sc_hist(dst2d)
    h, ah64 = _tc_mlp(xp, W1, b1, W2, b2)
    ztlo, zthi, u32, htlo, hthi, din64 = _tc_scale(h, deg16)
    zt0f = jnp.concatenate([ztlo, zthi], axis=0)
    htf = jnp.concatenate([htlo, hthi], axis=0)
    sf = _sc_prop(zt0f, src2d, dst2d, u32, htf)
    return _tc_finish(sf, din64, ah64)[:n]


# speedup vs baseline: 2.2589x; 1.0155x over previous
import functools

import jax
import jax.numpy as jnp
from jax.experimental import pallas as pl
from jax.experimental.pallas import tpu as pltpu
from jax.experimental.pallas import tpu_sc as plsc

ALPHA = 0.1
K = 10
NS = 16
EB = 128
NCH = 160
RSUB = 128
NPAD = 10240
CH = 32

_SC_PARAMS = pltpu.CompilerParams(use_tc_tiling_on_sc=False)
_MESH = dict(core_axis_name="c", subcore_axis_name="s")


def _sc_hist(dst2d):
    hpt = NCH // 2
    rpt = NPAD // NS

    @functools.partial(
        pl.kernel,
        out_type=jax.ShapeDtypeStruct((2, NPAD, 16), jnp.float32),
        mesh=plsc.VectorSubcoreMesh(**_MESH),
        compiler_params=_SC_PARAMS,
        scratch_types=[
            pltpu.VMEM_SHARED((NPAD, 16), jnp.float32),
            pltpu.VMEM((rpt, 16), jnp.float32),
            pltpu.VMEM((EB, 16), jnp.float32),
            pltpu.VMEM((EB,), jnp.int32),
        ],
    )
    def hist_kernel(dst_hbm, out_hbm, acc, rowbuf, ones_b, didx):
        core = jax.lax.axis_index("c")
        tile = jax.lax.axis_index("s")
        zeros16 = jnp.zeros((16,), jnp.float32)
        ones16 = jnp.ones((16,), jnp.float32)

        @pl.loop(0, rpt)
        def _(j):
            rowbuf[j, :] = zeros16

        pltpu.sync_copy(rowbuf, acc.at[pl.ds(tile * rpt, rpt)])

        @pl.loop(0, EB)
        def _(j):
            ones_b[j, :] = ones16

        plsc.subcore_barrier()
        c0 = tile * NCH + core * hpt

        @pl.loop(0, hpt)
        def _(i):
            pltpu.sync_copy(dst_hbm.at[c0 + i], didx)
            pltpu.sync_copy(ones_b, acc.at[didx], add=True)

        plsc.subcore_barrier()
        pltpu.sync_copy(acc.at[pl.ds(tile * rpt, rpt)], rowbuf)
        pltpu.sync_copy(rowbuf, out_hbm.at[core].at[pl.ds(tile * rpt, rpt)])

    return hist_kernel(dst2d)


def _tc_mlp(x, W1, b1, W2, b2):
    n = x.shape[0]
    c = W2.shape[1]
    blk = 1280

    def body(x_ref, w1_ref, b1_ref, w2_ref, b2_ref, h_ref, ah_ref):
        h1 = jnp.maximum(
            jnp.dot(x_ref[...], w1_ref[...], preferred_element_type=jnp.float32)
            + b1_ref[...], 0.0)
        h = jnp.dot(h1, w2_ref[...], preferred_element_type=jnp.float32) + b2_ref[...]
        h_ref[...] = h
        ah_ref[...] = ALPHA * h

    f = jnp.float32
    row = lambda i: (i, 0)
    bs_c = pl.BlockSpec((blk, c), row)
    return pl.pallas_call(
        body,
        grid=(n // blk,),
        in_specs=[
            pl.BlockSpec((blk, x.shape[1]), row),
            pl.BlockSpec(W1.shape, lambda i: (0, 0)),
            pl.BlockSpec((1, b1.shape[0]), lambda i: (0, 0)),
            pl.BlockSpec(W2.shape, lambda i: (0, 0)),
            pl.BlockSpec((1, b2.shape[0]), lambda i: (0, 0)),
        ],
        out_specs=[bs_c, bs_c],
        out_shape=(jax.ShapeDtypeStruct((n, c), f),
                   jax.ShapeDtypeStruct((n, c), f)),
    )(x, W1, b1.reshape(1, -1), W2, b2.reshape(1, -1))


def _tc_scale(h, deg16):
    n, c = h.shape
    blk = 1280

    def body(h_ref, deg_ref, ztlo_ref, zthi_ref, u_ref, htlo_ref, hthi_ref,
             din_ref):
        h = h_ref[...]
        deg = deg_ref[...][0, :, 0:1] + deg_ref[...][1, :, 0:1] + 1.0
        dinv = jax.lax.rsqrt(deg)
        zt = dinv * h
        ztlo_ref[...] = zt[:, :CH]
        zthi_ref[...] = zt[:, CH:]
        htlo_ref[...] = ALPHA * zt[:, :CH]
        hthi_ref[...] = ALPHA * zt[:, CH:]
        u_ref[...] = jnp.broadcast_to((1.0 - ALPHA) * dinv * dinv, (blk, CH))
        din_ref[...] = jnp.broadcast_to((1.0 - ALPHA) * dinv, (blk, c))

    f = jnp.float32
    row = lambda i: (i, 0)
    bs_h = pl.BlockSpec((blk, CH), row)
    bs_c = pl.BlockSpec((blk, c), row)
    return pl.pallas_call(
        body,
        grid=(n // blk,),
        in_specs=[bs_c, pl.BlockSpec((2, blk, 16), lambda i: (0, i, 0))],
        out_specs=[bs_h, bs_h, bs_h, bs_h, bs_h, bs_c],
        out_shape=(
            jax.ShapeDtypeStruct((n, CH), f),
            jax.ShapeDtypeStruct((n, CH), f),
            jax.ShapeDtypeStruct((n, CH), f),
            jax.ShapeDtypeStruct((n, CH), f),
            jax.ShapeDtypeStruct((n, CH), f),
            jax.ShapeDtypeStruct((n, c), f),
        ),
    )(h, deg16)


def _sc_prop(zt0f, src2d, dst2d, u32, htf):
    rpt = NPAD // NS
    nsub = rpt // RSUB

    @functools.partial(
        pl.kernel,
        out_type=jax.ShapeDtypeStruct((2 * NPAD, CH), jnp.float32),
        mesh=plsc.VectorSubcoreMesh(**_MESH),
        compiler_params=_SC_PARAMS,
        scratch_types=[
            pltpu.VMEM_SHARED((NPAD, CH), jnp.float32),
            pltpu.VMEM_SHARED((NPAD, CH), jnp.float32),
            pltpu.VMEM((NCH * EB,), jnp.int32),
            pltpu.VMEM((NCH * EB,), jnp.int32),
            pltpu.VMEM((RSUB, CH), jnp.float32),
            pltpu.VMEM((RSUB, CH), jnp.float32),
            pltpu.VMEM((4 * EB, CH), jnp.float32),
            pltpu.VMEM((4 * EB, CH), jnp.float32),
            pltpu.VMEM((RSUB, CH), jnp.float32),
            pltpu.SemaphoreType.DMA,
            pltpu.SemaphoreType.DMA,
            pltpu.SemaphoreType.DMA,
            pltpu.SemaphoreType.DMA,
        ],
    )
    def prop_kernel(zt0_hbm, src_hbm, dst_hbm, u_hbm, ht_hbm, sout_hbm,
                    zt, agg, sidx, didx, ubuf, hbuf, rows_a, rows_b, abuf,
                    gsa, gsb, ssa, ssb):
        core = jax.lax.axis_index("c")
        tile = jax.lax.axis_index("s")
        r0 = tile * rpt
        h0 = core * NPAD + r0

        pltpu.sync_copy(src_hbm.at[pl.ds(tile * NCH * EB, NCH * EB)], sidx)
        pltpu.sync_copy(dst_hbm.at[pl.ds(tile * NCH * EB, NCH * EB)], didx)
        for sub in range(nsub):
            ssp = pl.ds(r0 + sub * RSUB, RSUB)
            pltpu.sync_copy(zt0_hbm.at[pl.ds(h0 + sub * RSUB, RSUB)], abuf)
            pltpu.sync_copy(abuf, zt.at[ssp])
            pltpu.sync_copy(abuf, agg.at[ssp])
        plsc.subcore_barrier()

        GB = 4 * EB

        def g_start(i, buf, sem):
            pltpu.async_copy(zt.at[sidx.at[pl.ds(i * GB, GB)]], buf, sem)

        def g_wait(buf, sem):
            pltpu.make_async_copy(zt.at[sidx.at[pl.ds(0, GB)]], buf, sem).wait()

        def s_start(i, buf, sem):
            pltpu.async_copy(buf, agg.at[didx.at[pl.ds(i * GB, GB)]], sem,
                             add=True)

        def s_wait(buf, sem):
            pltpu.make_async_copy(buf, agg.at[didx.at[pl.ds(0, GB)]], sem).wait()

        npairs = NCH // 8

        def edge_pass():
            g_start(0, rows_a, gsa)

            @pl.loop(0, npairs)
            def _(p):
                i0 = 2 * p
                g_wait(rows_a, gsa)

                @pl.when(p > 0)
                def _():
                    s_wait(rows_b, ssb)

                g_start(i0 + 1, rows_b, gsb)
                s_start(i0, rows_a, ssa)
                g_wait(rows_b, gsb)
                s_wait(rows_a, ssa)

                @pl.when(p < npairs - 1)
                def _():
                    g_start(i0 + 2, rows_a, gsa)

                s_start(i0 + 1, rows_b, ssb)

            s_wait(rows_b, ssb)

        @pl.loop(0, K - 1)
        def _(k):
            edge_pass()
            plsc.subcore_barrier()
            for sub in range(nsub):
                ssp = pl.ds(r0 + sub * RSUB, RSUB)
                pltpu.sync_copy(agg.at[ssp], abuf)
                pltpu.sync_copy(u_hbm.at[pl.ds(r0 + sub * RSUB, RSUB)], ubuf)
                pltpu.sync_copy(ht_hbm.at[pl.ds(h0 + sub * RSUB, RSUB)], hbuf)

                @pl.loop(0, RSUB)
                def _(j):
                    for cc in range(CH // 16):
                        csl = pl.ds(cc * 16, 16)
                        abuf[j, csl] = (ubuf[j, csl] * abuf[j, csl]
                                        + hbuf[j, csl])

                pltpu.sync_copy(abuf, zt.at[ssp])
                pltpu.sync_copy(abuf, agg.at[ssp])
            plsc.subcore_barrier()

        edge_pass()
        plsc.subcore_barrier()
        for sub in range(nsub):
            pltpu.sync_copy(agg.at[pl.ds(r0 + sub * RSUB, RSUB)], abuf)
            pltpu.sync_copy(abuf, sout_hbm.at[pl.ds(h0 + sub * RSUB, RSUB)])

    return prop_kernel(zt0f, src2d, dst2d, u32, htf)


def _tc_finish(sf, din64, ah64):
    n, c = din64.shape

    def body(s_ref, din_ref, ah_ref, o_ref):
        s = jnp.concatenate([s_ref[...][:n], s_ref[...][n:]], axis=1)
        z = din_ref[...] * s + ah_ref[...]
        m = jnp.max(z, axis=1, keepdims=True)
        lse = jnp.log(jnp.sum(jnp.exp(z - m), axis=1, keepdims=True)) + m
        o_ref[...] = z - lse

    return pl.pallas_call(
        body, out_shape=jax.ShapeDtypeStruct((n, c), jnp.float32),
    )(sf, din64, ah64)


def kernel(x, edge_index, W1, b1, W2, b2):
    n = x.shape[0]
    e = edge_index.shape[1]
    epad = NS * NCH * EB
    xp = jnp.pad(x, ((0, NPAD - n), (0, 0)))
    pad = jnp.full((epad - e,), NPAD - 1, jnp.int32)
    src2d = jnp.concatenate([edge_index[0], pad]).reshape(NS * NCH, EB)
    dst2d = jnp.concatenate([edge_index[1], pad]).reshape(NS * NCH, EB)
    deg16 = _sc_hist(dst2d)
    h, ah64 = _tc_mlp(xp, W1, b1, W2, b2)
    ztlo, zthi, u32, htlo, hthi, din64 = _tc_scale(h, deg16)
    zt0f = jnp.concatenate([ztlo, zthi], axis=0)
    htf = jnp.concatenate([htlo, hthi], axis=0)
    sf = _sc_prop(zt0f, src2d.reshape(-1), dst2d.reshape(-1), u32, htf)
    return _tc_finish(sf, din64, ah64)[:n]


# speedup vs baseline: 2.3380x; 1.0350x over previous
import functools

import jax
import jax.numpy as jnp
from jax.experimental import pallas as pl
from jax.experimental.pallas import tpu as pltpu
from jax.experimental.pallas import tpu_sc as plsc

ALPHA = 0.1
K = 10
NS = 16
EB = 128
NCH = 160
RSUB = 128
NPAD = 10240
CH = 32

_SC_PARAMS = pltpu.CompilerParams(use_tc_tiling_on_sc=False)
_MESH = dict(core_axis_name="c", subcore_axis_name="s")


def _sc_hist(dstf):
    GB = 4 * EB
    ept = NCH * EB // 2
    nops = ept // GB
    rpt = NPAD // NS

    @functools.partial(
        pl.kernel,
        out_type=jax.ShapeDtypeStruct((2, NPAD, 16), jnp.float32),
        mesh=plsc.VectorSubcoreMesh(**_MESH),
        compiler_params=_SC_PARAMS,
        scratch_types=[
            pltpu.VMEM_SHARED((NPAD, 16), jnp.float32),
            pltpu.VMEM((rpt, 16), jnp.float32),
            pltpu.VMEM((GB, 16), jnp.float32),
            pltpu.VMEM((ept,), jnp.int32),
            pltpu.SemaphoreType.DMA,
        ],
    )
    def hist_kernel(dst_hbm, out_hbm, acc, rowbuf, ones_b, didx, sem):
        core = jax.lax.axis_index("c")
        tile = jax.lax.axis_index("s")
        zeros16 = jnp.zeros((16,), jnp.float32)
        ones16 = jnp.ones((16,), jnp.float32)

        e0 = tile * (2 * ept) + core * ept
        pltpu.async_copy(dst_hbm.at[pl.ds(e0, ept)], didx, sem)

        @pl.loop(0, rpt)
        def _(j):
            rowbuf[j, :] = zeros16

        pltpu.sync_copy(rowbuf, acc.at[pl.ds(tile * rpt, rpt)])

        @pl.loop(0, GB)
        def _(j):
            ones_b[j, :] = ones16

        pltpu.make_async_copy(dst_hbm.at[pl.ds(e0, ept)], didx, sem).wait()
        plsc.subcore_barrier()

        @pl.loop(0, nops)
        def _(i):
            pltpu.async_copy(ones_b, acc.at[didx.at[pl.ds(i * GB, GB)]], sem,
                             add=True)

        @pl.loop(0, nops)
        def _(i):
            pltpu.make_async_copy(ones_b, acc.at[didx.at[pl.ds(0, GB)]],
                                  sem).wait()

        plsc.subcore_barrier()
        pltpu.sync_copy(acc.at[pl.ds(tile * rpt, rpt)], rowbuf)
        pltpu.sync_copy(rowbuf, out_hbm.at[core].at[pl.ds(tile * rpt, rpt)])

    return hist_kernel(dstf)


def _tc_mlp(x, W1, b1, W2, b2):
    n = x.shape[0]
    c = W2.shape[1]
    blk = 1280

    def body(x_ref, w1_ref, b1_ref, w2_ref, b2_ref, h_ref, ah_ref):
        h1 = jnp.maximum(
            jnp.dot(x_ref[...], w1_ref[...], preferred_element_type=jnp.float32)
            + b1_ref[...], 0.0)
        h = jnp.dot(h1, w2_ref[...], preferred_element_type=jnp.float32) + b2_ref[...]
        h_ref[...] = h
        ah_ref[...] = ALPHA * h

    f = jnp.float32
    row = lambda i: (i, 0)
    bs_c = pl.BlockSpec((blk, c), row)
    return pl.pallas_call(
        body,
        grid=(n // blk,),
        in_specs=[
            pl.BlockSpec((blk, x.shape[1]), row),
            pl.BlockSpec(W1.shape, lambda i: (0, 0)),
            pl.BlockSpec((1, b1.shape[0]), lambda i: (0, 0)),
            pl.BlockSpec(W2.shape, lambda i: (0, 0)),
            pl.BlockSpec((1, b2.shape[0]), lambda i: (0, 0)),
        ],
        out_specs=[bs_c, bs_c],
        out_shape=(jax.ShapeDtypeStruct((n, c), f),
                   jax.ShapeDtypeStruct((n, c), f)),
    )(x, W1, b1.reshape(1, -1), W2, b2.reshape(1, -1))


def _tc_scale(h, deg16):
    n, c = h.shape
    blk = 1280

    def body(h_ref, deg_ref, ztlo_ref, zthi_ref, u_ref, htlo_ref, hthi_ref,
             din_ref):
        h = h_ref[...]
        deg = deg_ref[...][0, :, 0:1] + deg_ref[...][1, :, 0:1] + 1.0
        dinv = jax.lax.rsqrt(deg)
        zt = dinv * h
        ztlo_ref[...] = zt[:, :CH]
        zthi_ref[...] = zt[:, CH:]
        htlo_ref[...] = ALPHA * zt[:, :CH]
        hthi_ref[...] = ALPHA * zt[:, CH:]
        u_ref[...] = jnp.broadcast_to((1.0 - ALPHA) * dinv * dinv, (blk, CH))
        din_ref[...] = jnp.broadcast_to((1.0 - ALPHA) * dinv, (blk, c))

    f = jnp.float32
    row = lambda i: (i, 0)
    bs_h = pl.BlockSpec((blk, CH), row)
    bs_c = pl.BlockSpec((blk, c), row)
    return pl.pallas_call(
        body,
        grid=(n // blk,),
        in_specs=[bs_c, pl.BlockSpec((2, blk, 16), lambda i: (0, i, 0))],
        out_specs=[bs_h, bs_h, bs_h, bs_h, bs_h, bs_c],
        out_shape=(
            jax.ShapeDtypeStruct((n, CH), f),
            jax.ShapeDtypeStruct((n, CH), f),
            jax.ShapeDtypeStruct((n, CH), f),
            jax.ShapeDtypeStruct((n, CH), f),
            jax.ShapeDtypeStruct((n, CH), f),
            jax.ShapeDtypeStruct((n, c), f),
        ),
    )(h, deg16)


def _sc_prop(zt0f, src2d, dst2d, u32, htf):
    rpt = NPAD // NS
    nsub = rpt // RSUB

    @functools.partial(
        pl.kernel,
        out_type=jax.ShapeDtypeStruct((2 * NPAD, CH), jnp.float32),
        mesh=plsc.VectorSubcoreMesh(**_MESH),
        compiler_params=_SC_PARAMS,
        scratch_types=[
            pltpu.VMEM_SHARED((NPAD, CH), jnp.float32),
            pltpu.VMEM_SHARED((NPAD, CH), jnp.float32),
            pltpu.VMEM((NCH * EB,), jnp.int32),
            pltpu.VMEM((NCH * EB,), jnp.int32),
            pltpu.VMEM((RSUB, CH), jnp.float32),
            pltpu.VMEM((RSUB, CH), jnp.float32),
            pltpu.VMEM((4 * EB, CH), jnp.float32),
            pltpu.VMEM((4 * EB, CH), jnp.float32),
            pltpu.VMEM((RSUB, CH), jnp.float32),
            pltpu.SemaphoreType.DMA,
            pltpu.SemaphoreType.DMA,
            pltpu.SemaphoreType.DMA,
            pltpu.SemaphoreType.DMA,
        ],
    )
    def prop_kernel(zt0_hbm, src_hbm, dst_hbm, u_hbm, ht_hbm, sout_hbm,
                    zt, agg, sidx, didx, ubuf, hbuf, rows_a, rows_b, abuf,
                    gsa, gsb, ssa, ssb):
        core = jax.lax.axis_index("c")
        tile = jax.lax.axis_index("s")
        r0 = tile * rpt
        h0 = core * NPAD + r0

        pltpu.sync_copy(src_hbm.at[pl.ds(tile * NCH * EB, NCH * EB)], sidx)
        pltpu.sync_copy(dst_hbm.at[pl.ds(tile * NCH * EB, NCH * EB)], didx)
        for sub in range(nsub):
            ssp = pl.ds(r0 + sub * RSUB, RSUB)
            pltpu.sync_copy(zt0_hbm.at[pl.ds(h0 + sub * RSUB, RSUB)], abuf)
            pltpu.sync_copy(abuf, zt.at[ssp])
            pltpu.sync_copy(abuf, agg.at[ssp])
        plsc.subcore_barrier()

        GB = 4 * EB

        def g_start(i, buf, sem):
            pltpu.async_copy(zt.at[sidx.at[pl.ds(i * GB, GB)]], buf, sem)

        def g_wait(buf, sem):
            pltpu.make_async_copy(zt.at[sidx.at[pl.ds(0, GB)]], buf, sem).wait()

        def s_start(i, buf, sem):
            pltpu.async_copy(buf, agg.at[didx.at[pl.ds(i * GB, GB)]], sem,
                             add=True)

        def s_wait(buf, sem):
            pltpu.make_async_copy(buf, agg.at[didx.at[pl.ds(0, GB)]], sem).wait()

        npairs = NCH // 8

        def edge_pass():
            g_start(0, rows_a, gsa)

            @pl.loop(0, npairs)
            def _(p):
                i0 = 2 * p
                g_wait(rows_a, gsa)

                @pl.when(p > 0)
                def _():
                    s_wait(rows_b, ssb)

                g_start(i0 + 1, rows_b, gsb)
                s_start(i0, rows_a, ssa)
                g_wait(rows_b, gsb)
                s_wait(rows_a, ssa)

                @pl.when(p < npairs - 1)
                def _():
                    g_start(i0 + 2, rows_a, gsa)

                s_start(i0 + 1, rows_b, ssb)

            s_wait(rows_b, ssb)

        @pl.loop(0, K - 1)
        def _(k):
            edge_pass()
            plsc.subcore_barrier()
            for sub in range(nsub):
                ssp = pl.ds(r0 + sub * RSUB, RSUB)
                pltpu.sync_copy(agg.at[ssp], abuf)
                pltpu.sync_copy(u_hbm.at[pl.ds(r0 + sub * RSUB, RSUB)], ubuf)
                pltpu.sync_copy(ht_hbm.at[pl.ds(h0 + sub * RSUB, RSUB)], hbuf)

                @pl.loop(0, RSUB)
                def _(j):
                    for cc in range(CH // 16):
                        csl = pl.ds(cc * 16, 16)
                        abuf[j, csl] = (ubuf[j, csl] * abuf[j, csl]
                                        + hbuf[j, csl])

                pltpu.sync_copy(abuf, zt.at[ssp])
                pltpu.sync_copy(abuf, agg.at[ssp])
            plsc.subcore_barrier()

        edge_pass()
        plsc.subcore_barrier()
        for sub in range(nsub):
            pltpu.sync_copy(agg.at[pl.ds(r0 + sub * RSUB, RSUB)], abuf)
            pltpu.sync_copy(abuf, sout_hbm.at[pl.ds(h0 + sub * RSUB, RSUB)])

    return prop_kernel(zt0f, src2d, dst2d, u32, htf)


def _tc_finish(sf, din64, ah64):
    n, c = din64.shape

    def body(s_ref, din_ref, ah_ref, o_ref):
        s = jnp.concatenate([s_ref[...][:n], s_ref[...][n:]], axis=1)
        z = din_ref[...] * s + ah_ref[...]
        m = jnp.max(z, axis=1, keepdims=True)
        lse = jnp.log(jnp.sum(jnp.exp(z - m), axis=1, keepdims=True)) + m
        o_ref[...] = z - lse

    return pl.pallas_call(
        body, out_shape=jax.ShapeDtypeStruct((n, c), jnp.float32),
    )(sf, din64, ah64)


def kernel(x, edge_index, W1, b1, W2, b2):
    n = x.shape[0]
    e = edge_index.shape[1]
    epad = NS * NCH * EB
    xp = jnp.pad(x, ((0, NPAD - n), (0, 0)))
    pad = jnp.full((epad - e,), NPAD - 1, jnp.int32)
    src2d = jnp.concatenate([edge_index[0], pad]).reshape(NS * NCH, EB)
    dst2d = jnp.concatenate([edge_index[1], pad]).reshape(NS * NCH, EB)
    deg16 = _sc_hist(dst2d.reshape(-1))
    h, ah64 = _tc_mlp(xp, W1, b1, W2, b2)
    ztlo, zthi, u32, htlo, hthi, din64 = _tc_scale(h, deg16)
    zt0f = jnp.concatenate([ztlo, zthi], axis=0)
    htf = jnp.concatenate([htlo, hthi], axis=0)
    sf = _sc_prop(zt0f, src2d.reshape(-1), dst2d.reshape(-1), u32, htf)
    return _tc_finish(sf, din64, ah64)[:n]


# speedup vs baseline: 2.4401x; 1.0437x over previous
import functools

import jax
import jax.numpy as jnp
from jax.experimental import pallas as pl
from jax.experimental.pallas import tpu as pltpu
from jax.experimental.pallas import tpu_sc as plsc

ALPHA = 0.1
K = 10
NS = 16
EB = 128
NCH = 160
RSUB = 128
NPAD = 10240
CH = 32

_SC_PARAMS = pltpu.CompilerParams(use_tc_tiling_on_sc=False)
_MESH = dict(core_axis_name="c", subcore_axis_name="s")


def _sc_hist(dstf):
    GB = 4 * EB
    ept = NCH * EB // 2
    nops = ept // GB
    rpt = NPAD // NS

    @functools.partial(
        pl.kernel,
        out_type=jax.ShapeDtypeStruct((2, NPAD, 16), jnp.float32),
        mesh=plsc.VectorSubcoreMesh(**_MESH),
        compiler_params=_SC_PARAMS,
        scratch_types=[
            pltpu.VMEM_SHARED((NPAD, 16), jnp.float32),
            pltpu.VMEM((rpt, 16), jnp.float32),
            pltpu.VMEM((GB, 16), jnp.float32),
            pltpu.VMEM((ept,), jnp.int32),
            pltpu.SemaphoreType.DMA,
        ],
    )
    def hist_kernel(dst_hbm, out_hbm, acc, rowbuf, ones_b, didx, sem):
        core = jax.lax.axis_index("c")
        tile = jax.lax.axis_index("s")
        zeros16 = jnp.zeros((16,), jnp.float32)
        ones16 = jnp.ones((16,), jnp.float32)

        e0 = tile * (2 * ept) + core * ept
        pltpu.async_copy(dst_hbm.at[pl.ds(e0, ept)], didx, sem)

        @pl.loop(0, rpt)
        def _(j):
            rowbuf[j, :] = zeros16

        pltpu.sync_copy(rowbuf, acc.at[pl.ds(tile * rpt, rpt)])

        @pl.loop(0, GB)
        def _(j):
            ones_b[j, :] = ones16

        pltpu.make_async_copy(dst_hbm.at[pl.ds(e0, ept)], didx, sem).wait()
        plsc.subcore_barrier()

        @pl.loop(0, nops)
        def _(i):
            pltpu.async_copy(ones_b, acc.at[didx.at[pl.ds(i * GB, GB)]], sem,
                             add=True)

        @pl.loop(0, nops)
        def _(i):
            pltpu.make_async_copy(ones_b, acc.at[didx.at[pl.ds(0, GB)]],
                                  sem).wait()

        plsc.subcore_barrier()
        pltpu.sync_copy(acc.at[pl.ds(tile * rpt, rpt)], rowbuf)
        pltpu.sync_copy(rowbuf, out_hbm.at[core].at[pl.ds(tile * rpt, rpt)])

    return hist_kernel(dstf)


def _tc_mlp(x, W1, b1, W2, b2):
    n = x.shape[0]
    c = W2.shape[1]
    blk = 1280

    def body(x_ref, w1_ref, b1_ref, w2_ref, b2_ref, h_ref, ah_ref):
        h1 = jnp.maximum(
            jnp.dot(x_ref[...], w1_ref[...], preferred_element_type=jnp.float32)
            + b1_ref[...], 0.0)
        h = jnp.dot(h1, w2_ref[...], preferred_element_type=jnp.float32) + b2_ref[...]
        h_ref[...] = h
        ah_ref[...] = ALPHA * h

    f = jnp.float32
    row = lambda i: (i, 0)
    bs_c = pl.BlockSpec((blk, c), row)
    return pl.pallas_call(
        body,
        grid=(n // blk,),
        in_specs=[
            pl.BlockSpec((blk, x.shape[1]), row),
            pl.BlockSpec(W1.shape, lambda i: (0, 0)),
            pl.BlockSpec((1, b1.shape[0]), lambda i: (0, 0)),
            pl.BlockSpec(W2.shape, lambda i: (0, 0)),
            pl.BlockSpec((1, b2.shape[0]), lambda i: (0, 0)),
        ],
        out_specs=[bs_c, bs_c],
        out_shape=(jax.ShapeDtypeStruct((n, c), f),
                   jax.ShapeDtypeStruct((n, c), f)),
    )(x, W1, b1.reshape(1, -1), W2, b2.reshape(1, -1))


def _tc_scale(h, deg16):
    n, c = h.shape
    blk = 1280

    def body(h_ref, deg_ref, ztlo_ref, zthi_ref, u_ref, htlo_ref, hthi_ref,
             din_ref):
        h = h_ref[...]
        deg = deg_ref[...][0, :, 0:1] + deg_ref[...][1, :, 0:1] + 1.0
        dinv = jax.lax.rsqrt(deg)
        zt = dinv * h
        ztlo_ref[...] = zt[:, :CH]
        zthi_ref[...] = zt[:, CH:]
        htlo_ref[...] = ALPHA * zt[:, :CH]
        hthi_ref[...] = ALPHA * zt[:, CH:]
        u_ref[...] = jnp.broadcast_to((1.0 - ALPHA) * dinv * dinv, (blk, CH))
        din_ref[...] = jnp.broadcast_to((1.0 - ALPHA) * dinv, (blk, c))

    f = jnp.float32
    row = lambda i: (i, 0)
    bs_h = pl.BlockSpec((blk, CH), row)
    bs_c = pl.BlockSpec((blk, c), row)
    return pl.pallas_call(
        body,
        grid=(n // blk,),
        in_specs=[bs_c, pl.BlockSpec((2, blk, 16), lambda i: (0, i, 0))],
        out_specs=[bs_h, bs_h, bs_h, bs_h, bs_h, bs_c],
        out_shape=(
            jax.ShapeDtypeStruct((n, CH), f),
            jax.ShapeDtypeStruct((n, CH), f),
            jax.ShapeDtypeStruct((n, CH), f),
            jax.ShapeDtypeStruct((n, CH), f),
            jax.ShapeDtypeStruct((n, CH), f),
            jax.ShapeDtypeStruct((n, c), f),
        ),
    )(h, deg16)


def _sc_prop(zt0f, src2d, dst2d, u32, htf):
    rpt = NPAD // NS
    nsub = rpt // RSUB

    @functools.partial(
        pl.kernel,
        out_type=jax.ShapeDtypeStruct((2 * NPAD, CH), jnp.float32),
        mesh=plsc.VectorSubcoreMesh(**_MESH),
        compiler_params=_SC_PARAMS,
        scratch_types=[
            pltpu.VMEM_SHARED((NPAD, CH), jnp.float32),
            pltpu.VMEM_SHARED((NPAD, CH), jnp.float32),
            pltpu.VMEM((NCH * EB,), jnp.int32),
            pltpu.VMEM((NCH * EB,), jnp.int32),
            pltpu.VMEM((RSUB, CH), jnp.float32),
            pltpu.VMEM((RSUB, CH), jnp.float32),
            pltpu.VMEM((RSUB, CH), jnp.float32),
            pltpu.VMEM((RSUB, CH), jnp.float32),
            pltpu.VMEM((2 * EB, CH), jnp.float32),
            pltpu.VMEM((2 * EB, CH), jnp.float32),
            pltpu.VMEM((RSUB, CH), jnp.float32),
            pltpu.VMEM((RSUB, CH), jnp.float32),
            pltpu.SemaphoreType.DMA,
            pltpu.SemaphoreType.DMA,
            pltpu.SemaphoreType.DMA,
            pltpu.SemaphoreType.DMA,
            pltpu.SemaphoreType.DMA,
            pltpu.SemaphoreType.DMA,
        ],
    )
    def prop_kernel(zt0_hbm, src_hbm, dst_hbm, u_hbm, ht_hbm, sout_hbm,
                    zt, agg, sidx, didx, ubuf_a, hbuf_a, ubuf_b, hbuf_b,
                    rows_a, rows_b, abuf_a, abuf_b,
                    gsa, gsb, ssa, ssb, lsa, lsb):
        core = jax.lax.axis_index("c")
        tile = jax.lax.axis_index("s")
        r0 = tile * rpt
        h0 = core * NPAD + r0

        pltpu.sync_copy(src_hbm.at[pl.ds(tile * NCH * EB, NCH * EB)], sidx)
        pltpu.sync_copy(dst_hbm.at[pl.ds(tile * NCH * EB, NCH * EB)], didx)
        for sub in range(nsub):
            ssp = pl.ds(r0 + sub * RSUB, RSUB)
            pltpu.sync_copy(zt0_hbm.at[pl.ds(h0 + sub * RSUB, RSUB)], abuf_a)
            pltpu.sync_copy(abuf_a, zt.at[ssp])
            pltpu.sync_copy(abuf_a, agg.at[ssp])
        plsc.subcore_barrier()

        GB = 2 * EB

        def g_start(i, buf, sem):
            pltpu.async_copy(zt.at[sidx.at[pl.ds(i * GB, GB)]], buf, sem)

        def g_wait(buf, sem):
            pltpu.make_async_copy(zt.at[sidx.at[pl.ds(0, GB)]], buf, sem).wait()

        def s_start(i, buf, sem):
            pltpu.async_copy(buf, agg.at[didx.at[pl.ds(i * GB, GB)]], sem,
                             add=True)

        def s_wait(buf, sem):
            pltpu.make_async_copy(buf, agg.at[didx.at[pl.ds(0, GB)]], sem).wait()

        npairs = NCH // 4

        def edge_pass():
            g_start(0, rows_a, gsa)

            @pl.loop(0, npairs)
            def _(p):
                i0 = 2 * p
                g_wait(rows_a, gsa)

                @pl.when(p > 0)
                def _():
                    s_wait(rows_b, ssb)

                g_start(i0 + 1, rows_b, gsb)
                s_start(i0, rows_a, ssa)
                g_wait(rows_b, gsb)
                s_wait(rows_a, ssa)

                @pl.when(p < npairs - 1)
                def _():
                    g_start(i0 + 2, rows_a, gsa)

                s_start(i0 + 1, rows_b, ssb)

            s_wait(rows_b, ssb)

        @pl.loop(0, K - 1)
        def _(k):
            edge_pass()
            plsc.subcore_barrier()
            bufs = ((abuf_a, ubuf_a, hbuf_a, lsa), (abuf_b, ubuf_b, hbuf_b, lsb))

            def ew_load(sub, bs):
                ab, ub, hb, sem = bs
                pltpu.async_copy(u_hbm.at[pl.ds(r0 + sub * RSUB, RSUB)], ub, sem)
                pltpu.async_copy(ht_hbm.at[pl.ds(h0 + sub * RSUB, RSUB)], hb, sem)

            def ew_wait(bs):
                ab, ub, hb, sem = bs
                pltpu.make_async_copy(u_hbm.at[pl.ds(r0, RSUB)], ub, sem).wait()
                pltpu.make_async_copy(ht_hbm.at[pl.ds(h0, RSUB)], hb, sem).wait()

            ew_load(0, bufs[0])
            for sub in range(nsub):
                bs = bufs[sub % 2]
                ab, ub, hb, _ = bs
                pltpu.sync_copy(agg.at[pl.ds(r0 + sub * RSUB, RSUB)], ab)
                ew_wait(bs)
                if sub < nsub - 1:
                    ew_load(sub + 1, bufs[(sub + 1) % 2])

                @pl.loop(0, RSUB)
                def _(j):
                    for cc in range(CH // 16):
                        csl = pl.ds(cc * 16, 16)
                        ab[j, csl] = ub[j, csl] * ab[j, csl] + hb[j, csl]

                ssp = pl.ds(r0 + sub * RSUB, RSUB)
                pltpu.sync_copy(ab, zt.at[ssp])
                pltpu.sync_copy(ab, agg.at[ssp])
            plsc.subcore_barrier()

        edge_pass()
        plsc.subcore_barrier()
        for sub in range(nsub):
            pltpu.sync_copy(agg.at[pl.ds(r0 + sub * RSUB, RSUB)], abuf_a)
            pltpu.sync_copy(abuf_a, sout_hbm.at[pl.ds(h0 + sub * RSUB, RSUB)])

    return prop_kernel(zt0f, src2d, dst2d, u32, htf)


def _tc_finish(sf, din64, ah64):
    n, c = din64.shape

    def body(s_ref, din_ref, ah_ref, o_ref):
        s = jnp.concatenate([s_ref[...][:n], s_ref[...][n:]], axis=1)
        z = din_ref[...] * s + ah_ref[...]
        m = jnp.max(z, axis=1, keepdims=True)
        lse = jnp.log(jnp.sum(jnp.exp(z - m), axis=1, keepdims=True)) + m
        o_ref[...] = z - lse

    return pl.pallas_call(
        body, out_shape=jax.ShapeDtypeStruct((n, c), jnp.float32),
    )(sf, din64, ah64)


def kernel(x, edge_index, W1, b1, W2, b2):
    n = x.shape[0]
    e = edge_index.shape[1]
    epad = NS * NCH * EB
    xp = jnp.pad(x, ((0, NPAD - n), (0, 0)))
    pad = jnp.full((epad - e,), NPAD - 1, jnp.int32)
    src2d = jnp.concatenate([edge_index[0], pad]).reshape(NS * NCH, EB)
    dst2d = jnp.concatenate([edge_index[1], pad]).reshape(NS * NCH, EB)
    deg16 = _sc_hist(dst2d.reshape(-1))
    h, ah64 = _tc_mlp(xp, W1, b1, W2, b2)
    ztlo, zthi, u32, htlo, hthi, din64 = _tc_scale(h, deg16)
    zt0f = jnp.concatenate([ztlo, zthi], axis=0)
    htf = jnp.concatenate([htlo, hthi], axis=0)
    sf = _sc_prop(zt0f, src2d.reshape(-1), dst2d.reshape(-1), u32, htf)
    return _tc_finish(sf, din64, ah64)[:n]


# speedup vs baseline: 2.5056x; 1.0269x over previous
import functools

import jax
import jax.numpy as jnp
from jax.experimental import pallas as pl
from jax.experimental.pallas import tpu as pltpu
from jax.experimental.pallas import tpu_sc as plsc

ALPHA = 0.1
K = 10
NS = 16
EB = 128
NCH = 160
RSUB = 128
NPAD = 10240
CH = 32

_SC_PARAMS = pltpu.CompilerParams(use_tc_tiling_on_sc=False)
_MESH = dict(core_axis_name="c", subcore_axis_name="s")


def _sc_hist(dstf):
    GB = 4 * EB
    ept = NCH * EB // 2
    nops = ept // GB
    rpt = NPAD // NS

    @functools.partial(
        pl.kernel,
        out_type=jax.ShapeDtypeStruct((2, NPAD, 16), jnp.float32),
        mesh=plsc.VectorSubcoreMesh(**_MESH),
        compiler_params=_SC_PARAMS,
        scratch_types=[
            pltpu.VMEM_SHARED((NPAD, 16), jnp.float32),
            pltpu.VMEM((rpt, 16), jnp.float32),
            pltpu.VMEM((GB, 16), jnp.float32),
            pltpu.VMEM((ept,), jnp.int32),
            pltpu.SemaphoreType.DMA,
        ],
    )
    def hist_kernel(dst_hbm, out_hbm, acc, rowbuf, ones_b, didx, sem):
        core = jax.lax.axis_index("c")
        tile = jax.lax.axis_index("s")
        zeros16 = jnp.zeros((16,), jnp.float32)
        ones16 = jnp.ones((16,), jnp.float32)

        e0 = tile * (2 * ept) + core * ept
        pltpu.async_copy(dst_hbm.at[pl.ds(e0, ept)], didx, sem)

        @pl.loop(0, rpt)
        def _(j):
            rowbuf[j, :] = zeros16

        pltpu.sync_copy(rowbuf, acc.at[pl.ds(tile * rpt, rpt)])

        @pl.loop(0, GB)
        def _(j):
            ones_b[j, :] = ones16

        pltpu.make_async_copy(dst_hbm.at[pl.ds(e0, ept)], didx, sem).wait()
        plsc.subcore_barrier()

        @pl.loop(0, nops)
        def _(i):
            pltpu.async_copy(ones_b, acc.at[didx.at[pl.ds(i * GB, GB)]], sem,
                             add=True)

        @pl.loop(0, nops)
        def _(i):
            pltpu.make_async_copy(ones_b, acc.at[didx.at[pl.ds(0, GB)]],
                                  sem).wait()

        plsc.subcore_barrier()
        pltpu.sync_copy(acc.at[pl.ds(tile * rpt, rpt)], rowbuf)
        pltpu.sync_copy(rowbuf, out_hbm.at[core].at[pl.ds(tile * rpt, rpt)])

    return hist_kernel(dstf)


def _tc_mlp(x, W1, b1, W2, b2):
    n = x.shape[0]
    c = W2.shape[1]
    blk = 1280

    def body(x_ref, w1_ref, b1_ref, w2_ref, b2_ref, h_ref, ah_ref):
        h1 = jnp.maximum(
            jnp.dot(x_ref[...], w1_ref[...], preferred_element_type=jnp.float32)
            + b1_ref[...], 0.0)
        h = jnp.dot(h1, w2_ref[...], preferred_element_type=jnp.float32) + b2_ref[...]
        h_ref[...] = h
        ah_ref[...] = ALPHA * h

    f = jnp.float32
    row = lambda i: (i, 0)
    bs_c = pl.BlockSpec((blk, c), row)
    return pl.pallas_call(
        body,
        grid=(n // blk,),
        in_specs=[
            pl.BlockSpec((blk, x.shape[1]), row),
            pl.BlockSpec(W1.shape, lambda i: (0, 0)),
            pl.BlockSpec((1, b1.shape[0]), lambda i: (0, 0)),
            pl.BlockSpec(W2.shape, lambda i: (0, 0)),
            pl.BlockSpec((1, b2.shape[0]), lambda i: (0, 0)),
        ],
        out_specs=[bs_c, bs_c],
        out_shape=(jax.ShapeDtypeStruct((n, c), f),
                   jax.ShapeDtypeStruct((n, c), f)),
    )(x, W1, b1.reshape(1, -1), W2, b2.reshape(1, -1))


def _tc_scale(h, deg16):
    n, c = h.shape
    blk = 1280
    nb = n // blk

    def body(h_ref, deg_ref, zt_ref, u_ref, ht_ref, din_ref):
        j = pl.program_id(0)
        h = h_ref[...]
        deg = deg_ref[...][0, :, 0:1] + deg_ref[...][1, :, 0:1] + 1.0
        dinv = jax.lax.rsqrt(deg)
        zt = dinv * h
        half = jnp.where(j == 0, zt[:, :CH], zt[:, CH:])
        zt_ref[...] = half
        ht_ref[...] = ALPHA * half
        u_ref[...] = jnp.broadcast_to((1.0 - ALPHA) * dinv * dinv, (blk, CH))
        din_ref[...] = jnp.broadcast_to((1.0 - ALPHA) * dinv, (blk, c))

    f = jnp.float32
    return pl.pallas_call(
        body,
        grid=(2, nb),
        in_specs=[pl.BlockSpec((blk, c), lambda j, i: (i, 0)),
                  pl.BlockSpec((2, blk, 16), lambda j, i: (0, i, 0))],
        out_specs=[
            pl.BlockSpec((blk, CH), lambda j, i: (j * nb + i, 0)),
            pl.BlockSpec((blk, CH), lambda j, i: (i, 0)),
            pl.BlockSpec((blk, CH), lambda j, i: (j * nb + i, 0)),
            pl.BlockSpec((blk, c), lambda j, i: (i, 0)),
        ],
        out_shape=(
            jax.ShapeDtypeStruct((2 * n, CH), f),
            jax.ShapeDtypeStruct((n, CH), f),
            jax.ShapeDtypeStruct((2 * n, CH), f),
            jax.ShapeDtypeStruct((n, c), f),
        ),
    )(h, deg16)


def _sc_prop(zt0f, src2d, dst2d, u32, htf):
    rpt = NPAD // NS
    nsub = rpt // RSUB

    @functools.partial(
        pl.kernel,
        out_type=jax.ShapeDtypeStruct((2 * NPAD, CH), jnp.float32),
        mesh=plsc.VectorSubcoreMesh(**_MESH),
        compiler_params=_SC_PARAMS,
        scratch_types=[
            pltpu.VMEM_SHARED((NPAD, CH), jnp.float32),
            pltpu.VMEM_SHARED((NPAD, CH), jnp.float32),
            pltpu.VMEM((NCH * EB,), jnp.int32),
            pltpu.VMEM((NCH * EB,), jnp.int32),
            pltpu.VMEM((RSUB, CH), jnp.float32),
            pltpu.VMEM((RSUB, CH), jnp.float32),
            pltpu.VMEM((RSUB, CH), jnp.float32),
            pltpu.VMEM((RSUB, CH), jnp.float32),
            pltpu.VMEM((2 * EB, CH), jnp.float32),
            pltpu.VMEM((2 * EB, CH), jnp.float32),
            pltpu.VMEM((RSUB, CH), jnp.float32),
            pltpu.VMEM((RSUB, CH), jnp.float32),
            pltpu.SemaphoreType.DMA,
            pltpu.SemaphoreType.DMA,
            pltpu.SemaphoreType.DMA,
            pltpu.SemaphoreType.DMA,
            pltpu.SemaphoreType.DMA,
            pltpu.SemaphoreType.DMA,
        ],
    )
    def prop_kernel(zt0_hbm, src_hbm, dst_hbm, u_hbm, ht_hbm, sout_hbm,
                    zt, agg, sidx, didx, ubuf_a, hbuf_a, ubuf_b, hbuf_b,
                    rows_a, rows_b, abuf_a, abuf_b,
                    gsa, gsb, ssa, ssb, lsa, lsb):
        core = jax.lax.axis_index("c")
        tile = jax.lax.axis_index("s")
        r0 = tile * rpt
        h0 = core * NPAD + r0

        pltpu.sync_copy(src_hbm.at[pl.ds(tile * NCH * EB, NCH * EB)], sidx)
        pltpu.sync_copy(dst_hbm.at[pl.ds(tile * NCH * EB, NCH * EB)], didx)
        for sub in range(nsub):
            ssp = pl.ds(r0 + sub * RSUB, RSUB)
            pltpu.sync_copy(zt0_hbm.at[pl.ds(h0 + sub * RSUB, RSUB)], abuf_a)
            pltpu.sync_copy(abuf_a, zt.at[ssp])
            pltpu.sync_copy(abuf_a, agg.at[ssp])
        plsc.subcore_barrier()

        GB = 2 * EB

        def g_start(i, buf, sem):
            pltpu.async_copy(zt.at[sidx.at[pl.ds(i * GB, GB)]], buf, sem)

        def g_wait(buf, sem):
            pltpu.make_async_copy(zt.at[sidx.at[pl.ds(0, GB)]], buf, sem).wait()

        def s_start(i, buf, sem):
            pltpu.async_copy(buf, agg.at[didx.at[pl.ds(i * GB, GB)]], sem,
                             add=True)

        def s_wait(buf, sem):
            pltpu.make_async_copy(buf, agg.at[didx.at[pl.ds(0, GB)]], sem).wait()

        npairs = NCH // 4

        def edge_pass():
            g_start(0, rows_a, gsa)

            @pl.loop(0, npairs)
            def _(p):
                i0 = 2 * p
                g_wait(rows_a, gsa)

                @pl.when(p > 0)
                def _():
                    s_wait(rows_b, ssb)

                g_start(i0 + 1, rows_b, gsb)
                s_start(i0, rows_a, ssa)
                g_wait(rows_b, gsb)
                s_wait(rows_a, ssa)

                @pl.when(p < npairs - 1)
                def _():
                    g_start(i0 + 2, rows_a, gsa)

                s_start(i0 + 1, rows_b, ssb)

            s_wait(rows_b, ssb)

        @pl.loop(0, K - 1)
        def _(k):
            edge_pass()
            plsc.subcore_barrier()
            bufs = ((abuf_a, ubuf_a, hbuf_a, lsa), (abuf_b, ubuf_b, hbuf_b, lsb))

            def ew_load(sub, bs):
                ab, ub, hb, sem = bs
                pltpu.async_copy(u_hbm.at[pl.ds(r0 + sub * RSUB, RSUB)], ub, sem)
                pltpu.async_copy(ht_hbm.at[pl.ds(h0 + sub * RSUB, RSUB)], hb, sem)

            def ew_wait(bs):
                ab, ub, hb, sem = bs
                pltpu.make_async_copy(u_hbm.at[pl.ds(r0, RSUB)], ub, sem).wait()
                pltpu.make_async_copy(ht_hbm.at[pl.ds(h0, RSUB)], hb, sem).wait()

            ew_load(0, bufs[0])
            for sub in range(nsub):
                bs = bufs[sub % 2]
                ab, ub, hb, _ = bs
                pltpu.sync_copy(agg.at[pl.ds(r0 + sub * RSUB, RSUB)], ab)
                ew_wait(bs)
                if sub < nsub - 1:
                    ew_load(sub + 1, bufs[(sub + 1) % 2])

                @pl.loop(0, RSUB)
                def _(j):
                    for cc in range(CH // 16):
                        csl = pl.ds(cc * 16, 16)
                        ab[j, csl] = ub[j, csl] * ab[j, csl] + hb[j, csl]

                ssp = pl.ds(r0 + sub * RSUB, RSUB)
                pltpu.sync_copy(ab, zt.at[ssp])
                pltpu.sync_copy(ab, agg.at[ssp])
            plsc.subcore_barrier()

        edge_pass()
        plsc.subcore_barrier()
        for sub in range(nsub):
            pltpu.sync_copy(agg.at[pl.ds(r0 + sub * RSUB, RSUB)], abuf_a)
            pltpu.sync_copy(abuf_a, sout_hbm.at[pl.ds(h0 + sub * RSUB, RSUB)])

    return prop_kernel(zt0f, src2d, dst2d, u32, htf)


def _tc_finish(sf, din64, ah64, n):
    c = ah64.shape[1]
    blk = 2000
    s3 = sf.reshape(2, NPAD, CH)

    def body(slo_ref, shi_ref, din_ref, ah_ref, o_ref):
        s = jnp.concatenate([slo_ref[...][0], shi_ref[...][0]], axis=1)
        z = din_ref[...] * s + ah_ref[...]
        m = jnp.max(z, axis=1, keepdims=True)
        lse = jnp.log(jnp.sum(jnp.exp(z - m), axis=1, keepdims=True)) + m
        o_ref[...] = z - lse

    return pl.pallas_call(
        body,
        grid=(n // blk,),
        in_specs=[pl.BlockSpec((1, blk, CH), lambda i: (0, i, 0)),
                  pl.BlockSpec((1, blk, CH), lambda i: (1, i, 0)),
                  pl.BlockSpec((blk, c), lambda i: (i, 0)),
                  pl.BlockSpec((blk, c), lambda i: (i, 0))],
        out_specs=pl.BlockSpec((blk, c), lambda i: (i, 0)),
        out_shape=jax.ShapeDtypeStruct((n, c), jnp.float32),
    )(s3, s3, din64, ah64)


def kernel(x, edge_index, W1, b1, W2, b2):
    n = x.shape[0]
    e = edge_index.shape[1]
    epad = NS * NCH * EB
    xp = jnp.pad(x, ((0, NPAD - n), (0, 0)))
    pad = jnp.full((epad - e,), NPAD - 1, jnp.int32)
    src2d = jnp.concatenate([edge_index[0], pad]).reshape(NS * NCH, EB)
    dst2d = jnp.concatenate([edge_index[1], pad]).reshape(NS * NCH, EB)
    deg16 = _sc_hist(dst2d.reshape(-1))
    h, ah64 = _tc_mlp(xp, W1, b1, W2, b2)
    zt0f, u32, htf, din64 = _tc_scale(h, deg16)
    sf = _sc_prop(zt0f, src2d.reshape(-1), dst2d.reshape(-1), u32, htf)
    return _tc_finish(sf, din64, ah64, n)
